# baseline (speedup 1.0000x reference)
import functools

import jax
import jax.numpy as jnp
from jax import lax
from jax.experimental import pallas as pl
from jax.experimental.pallas import tpu as pltpu
from jax.experimental.pallas import tpu_sc as plsc

N_VOCAB = 100000
N_EMBD = 768
N_TOKENS = 2048
BATCH = 4

ROWS = BATCH * N_TOKENS
NC = 2
NS = 16
L = 16
NW = NC * NS
ROWS_PER_W = ROWS // NW
CHUNK = 64
NCHUNK = ROWS_PER_W // CHUNK
VPR = N_EMBD // L

_mesh = plsc.VectorSubcoreMesh(core_axis_name="c", subcore_axis_name="s")


@functools.partial(
    pl.kernel,
    mesh=_mesh,
    out_type=jax.ShapeDtypeStruct((ROWS, N_EMBD), jnp.float32),
    scratch_types=[
        pltpu.VMEM((ROWS_PER_W,), jnp.int32),
        pltpu.VMEM((CHUNK, N_EMBD), jnp.float32),
        pltpu.VMEM((CHUNK, N_EMBD), jnp.float32),
        pltpu.SemaphoreType.DMA,
    ],
)
def _embed(tokens_hbm, table_hbm, pos_hbm, out_hbm, idx_v, rows_v, pos_v, sem):
    wid = lax.axis_index("s") * NC + lax.axis_index("c")
    base = wid * ROWS_PER_W
    t0 = base % N_TOKENS
    pltpu.sync_copy(tokens_hbm.at[pl.ds(base, ROWS_PER_W)], idx_v)
    for c in range(NCHUNK):
        pltpu.async_copy(
            table_hbm.at[idx_v.at[pl.ds(c * CHUNK, CHUNK)]], rows_v, sem
        ).wait()
        pltpu.sync_copy(pos_hbm.at[pl.ds(t0 + c * CHUNK, CHUNK)], pos_v)

        def body(i, _):
            def inner(j, _):
                sl = pl.ds(j * L, L)
                rows_v[i, sl] = rows_v[i, sl] + pos_v[i, sl]
                return 0

            return lax.fori_loop(0, VPR, inner, 0)

        lax.fori_loop(0, CHUNK, body, 0)
        pltpu.sync_copy(rows_v, out_hbm.at[pl.ds(base + c * CHUNK, CHUNK)])


def kernel(tokens, token_embedding, pos_embedding):
    flat = tokens.reshape(-1).astype(jnp.int32)
    out = _embed(flat, token_embedding, pos_embedding)
    return out.reshape(BATCH, N_TOKENS, N_EMBD)


# speedup vs baseline: 2.0284x; 2.0284x over previous
import functools

import jax
import jax.numpy as jnp
from jax import lax
from jax.experimental import pallas as pl
from jax.experimental.pallas import tpu as pltpu
from jax.experimental.pallas import tpu_sc as plsc

N_VOCAB = 100000
N_EMBD = 768
N_TOKENS = 2048
BATCH = 4

ROWS = BATCH * N_TOKENS
NC = 2
NS = 16
L = 16
NW = NC * NS
ROWS_PER_W = ROWS // NW
CHUNK = 32
NCHUNK = ROWS_PER_W // CHUNK
VPR = N_EMBD // L

_mesh = plsc.VectorSubcoreMesh(core_axis_name="c", subcore_axis_name="s")


@functools.partial(
    pl.kernel,
    mesh=_mesh,
    out_type=jax.ShapeDtypeStruct((ROWS, N_EMBD), jnp.float32),
    scratch_types=[
        pltpu.VMEM((ROWS_PER_W,), jnp.int32),
        pltpu.VMEM((CHUNK, N_EMBD), jnp.float32),
        pltpu.VMEM((CHUNK, N_EMBD), jnp.float32),
        pltpu.VMEM((CHUNK, N_EMBD), jnp.float32),
        pltpu.VMEM((CHUNK, N_EMBD), jnp.float32),
        pltpu.SemaphoreType.DMA,
        pltpu.SemaphoreType.DMA,
        pltpu.SemaphoreType.DMA,
        pltpu.SemaphoreType.DMA,
        pltpu.SemaphoreType.DMA,
        pltpu.SemaphoreType.DMA,
    ],
)
def _embed(tokens_hbm, table_hbm, pos_hbm, out_hbm,
           idx_v, rows0, rows1, pos0, pos1,
           gsem0, gsem1, psem0, psem1, osem0, osem1):
    wid = lax.axis_index("s") * NC + lax.axis_index("c")
    base = wid * ROWS_PER_W
    t0 = base % N_TOKENS
    rows = (rows0, rows1)
    pos = (pos0, pos1)
    gsem = (gsem0, gsem1)
    psem = (psem0, psem1)
    osem = (osem0, osem1)

    pltpu.sync_copy(tokens_hbm.at[pl.ds(base, ROWS_PER_W)], idx_v)

    def start_in(c):
        b = c % 2
        g = pltpu.async_copy(
            table_hbm.at[idx_v.at[pl.ds(c * CHUNK, CHUNK)]], rows[b], gsem[b])
        p = pltpu.async_copy(
            pos_hbm.at[pl.ds(t0 + c * CHUNK, CHUNK)], pos[b], psem[b])
        return g, p

    inflight = {0: start_in(0)}
    out_inflight = {}
    for c in range(NCHUNK):
        b = c % 2
        g, p = inflight.pop(c)
        g.wait()
        p.wait()
        if c + 1 < NCHUNK:
            if c - 1 in out_inflight:
                out_inflight.pop(c - 1).wait()
            inflight[c + 1] = start_in(c + 1)

        def body(i, _):
            for j in range(VPR):
                sl = pl.ds(j * L, L)
                rows[b][i, sl] = rows[b][i, sl] + pos[b][i, sl]
            return 0

        lax.fori_loop(0, CHUNK, body, 0)
        out_inflight[c] = pltpu.async_copy(
            rows[b], out_hbm.at[pl.ds(base + c * CHUNK, CHUNK)], osem[b])
    for c in list(out_inflight):
        out_inflight.pop(c).wait()


def kernel(tokens, token_embedding, pos_embedding):
    flat = tokens.reshape(-1).astype(jnp.int32)
    out = _embed(flat, token_embedding, pos_embedding)
    return out.reshape(BATCH, N_TOKENS, N_EMBD)
